# baseline (speedup 1.0000x reference)
import functools

import jax
import jax.numpy as jnp
from jax import lax
from jax.experimental import pallas as pl
from jax.experimental.pallas import tpu as pltpu
from jax.experimental.pallas import tpu_sc as plsc

B = 4
S = 2048
D = 768
MAX_POS = 2048
OFFSET = 2
N = B * S

NC, NS = 2, 16
NW = NC * NS
RPW = N // NW
CH = 32
NCH = RPW // CH
NBUF = 2

_mesh = plsc.VectorSubcoreMesh(core_axis_name="c", subcore_axis_name="s")


L = 16


@functools.partial(
    pl.kernel,
    mesh=_mesh,
    out_type=jax.ShapeDtypeStruct((N, D), jnp.float32),
    scratch_types=[
        pltpu.VMEM((B, RPW), jnp.int32),
        pltpu.VMEM_SHARED((NS, 2, CH, D), jnp.float32),
    ]
    + [pltpu.VMEM((CH, D), jnp.float32) for _ in range(NBUF)]
    + [pltpu.SemaphoreType.DMA for _ in range(3 * NBUF)],
)
def _tok_kernel(ids_hbm, table_hbm, out_tok, idx_v, shared, *bufs_and_sems):
    bufs = bufs_and_sems[:NBUF]
    gsems = bufs_and_sems[NBUF:2 * NBUF]
    csems = bufs_and_sems[2 * NBUF:3 * NBUF]
    ssems = bufs_and_sems[3 * NBUF:]
    wid = lax.axis_index("s") * NC + lax.axis_index("c")
    b = wid % B
    cb = wid // B
    base = b * S + cb * RPW

    pltpu.sync_copy(ids_hbm.at[:, pl.ds(cb * RPW, RPW)], idx_v)

    def gather(c):
        cps = []
        for k in range(CH // L):
            vec = idx_v[b, pl.ds(c * CH + k * L, L)]
            cp = pltpu.make_async_copy(
                table_hbm.at[vec],
                bufs[c % NBUF].at[pl.ds(k * L, L)],
                gsems[c % NBUF],
            )
            cp.start()
            cps.append(cp)
        return cps

    sid = lax.axis_index("s")

    def ccopy(c):
        return pltpu.make_async_copy(
            bufs[c % NBUF], shared.at[sid, c % 2], csems[c % NBUF])

    def store(c):
        return pltpu.make_async_copy(
            shared.at[sid, c % 2], out_tok.at[pl.ds(base + c * CH, CH)],
            ssems[c % NBUF])

    LA = 2
    pend = {c: gather(c) for c in range(min(LA, NCH))}
    copies = {}
    stores = {}
    for c in range(NCH):
        for cp in pend.pop(c):
            cp.wait()
        if c >= 2:
            stores.pop(c - 2).wait()
        copies[c] = ccopy(c)
        copies[c].start()
        copies[c].wait()
        stores[c] = store(c)
        stores[c].start()
        nxt = c + LA
        if nxt < NCH:
            pend[nxt] = gather(nxt)
    for c in sorted(stores):
        stores[c].wait()


def _pos_body(src_ref, out_ref, mask_ref):
    out_ref[0] = src_ref[pl.ds(OFFSET, S), :]

    @pl.when(pl.program_id(0) == 0)
    def _():
        mask_ref[...] = jnp.ones_like(mask_ref)


_pos_kernel = pl.pallas_call(
    _pos_body,
    grid=(B,),
    in_specs=[pl.BlockSpec((MAX_POS + OFFSET, D), lambda b: (0, 0))],
    out_specs=[
        pl.BlockSpec((1, S, D), lambda b: (b, 0, 0)),
        pl.BlockSpec((B, S), lambda b: (0, 0)),
    ],
    out_shape=[
        jax.ShapeDtypeStruct((B, S, D), jnp.float32),
        jax.ShapeDtypeStruct((B, S), jnp.float32),
    ],
)


def kernel(input_ids, embed_tokens, embed_positions):
    ids = input_ids.astype(jnp.int32)
    tok_flat = _tok_kernel(ids, embed_tokens)
    pos_embeds, attention_mask = _pos_kernel(embed_positions)
    inputs_embeds = tok_flat.reshape(B, S, D)
    return (inputs_embeds, attention_mask, pos_embeds)


# speedup vs baseline: 1.0140x; 1.0140x over previous
import functools

import jax
import jax.numpy as jnp
from jax import lax
from jax.experimental import pallas as pl
from jax.experimental.pallas import tpu as pltpu
from jax.experimental.pallas import tpu_sc as plsc

B = 4
S = 2048
D = 768
MAX_POS = 2048
OFFSET = 2
N = B * S

NC, NS = 2, 16
NW = NC * NS
RPW = N // NW
CH = 16
NCH = RPW // CH
NBUF = 10

_mesh = plsc.VectorSubcoreMesh(core_axis_name="c", subcore_axis_name="s")


L = 16


@functools.partial(
    pl.kernel,
    mesh=_mesh,
    out_type=jax.ShapeDtypeStruct((N, D), jnp.float32),
    scratch_types=[
        pltpu.VMEM((B, RPW), jnp.int32),
    ]
    + [pltpu.VMEM((CH, D), jnp.float32) for _ in range(NBUF)]
    + [pltpu.SemaphoreType.DMA for _ in range(2 * NBUF)],
)
def _tok_kernel(ids_hbm, table_hbm, out_tok, idx_v, *bufs_and_sems):
    bufs = bufs_and_sems[:NBUF]
    gsems = bufs_and_sems[NBUF:2 * NBUF]
    ssems = bufs_and_sems[2 * NBUF:]
    wid = lax.axis_index("s") * NC + lax.axis_index("c")
    b = wid % B
    cb = wid // B
    base = b * S + cb * RPW

    pltpu.sync_copy(ids_hbm.at[:, pl.ds(cb * RPW, RPW)], idx_v)

    def gather(c):
        cps = []
        for k in range(CH // L):
            vec = idx_v[b, pl.ds(c * CH + k * L, L)]
            cp = pltpu.make_async_copy(
                table_hbm.at[vec],
                bufs[c % NBUF].at[pl.ds(k * L, L)],
                gsems[c % NBUF],
            )
            cp.start()
            cps.append(cp)
        return cps

    def store(c):
        return pltpu.make_async_copy(
            bufs[c % NBUF], out_tok.at[pl.ds(base + c * CH, CH)],
            ssems[c % NBUF])

    LA = 5
    pend = {c: gather(c) for c in range(min(LA, NCH))}
    stores = {}
    for c in range(NCH):
        for cp in pend.pop(c):
            cp.wait()
        stores[c] = store(c)
        stores[c].start()
        nxt = c + LA
        if nxt < NCH:
            old = nxt - NBUF
            if old >= 0:
                stores.pop(old).wait()
            pend[nxt] = gather(nxt)
    for c in sorted(stores):
        stores[c].wait()


def _pos_body(src_ref, out_ref, mask_ref):
    out_ref[0] = src_ref[pl.ds(OFFSET, S), :]

    @pl.when(pl.program_id(0) == 0)
    def _():
        mask_ref[...] = jnp.ones_like(mask_ref)


_pos_kernel = pl.pallas_call(
    _pos_body,
    grid=(B,),
    in_specs=[pl.BlockSpec((MAX_POS + OFFSET, D), lambda b: (0, 0))],
    out_specs=[
        pl.BlockSpec((1, S, D), lambda b: (b, 0, 0)),
        pl.BlockSpec((B, S), lambda b: (0, 0)),
    ],
    out_shape=[
        jax.ShapeDtypeStruct((B, S, D), jnp.float32),
        jax.ShapeDtypeStruct((B, S), jnp.float32),
    ],
)


def kernel(input_ids, embed_tokens, embed_positions):
    ids = input_ids.astype(jnp.int32)
    tok_flat = _tok_kernel(ids, embed_tokens)
    pos_embeds, attention_mask = _pos_kernel(embed_positions)
    inputs_embeds = tok_flat.reshape(B, S, D)
    return (inputs_embeds, attention_mask, pos_embeds)


# speedup vs baseline: 1.0169x; 1.0029x over previous
import functools

import jax
import jax.numpy as jnp
from jax import lax
from jax.experimental import pallas as pl
from jax.experimental.pallas import tpu as pltpu
from jax.experimental.pallas import tpu_sc as plsc

B = 4
S = 2048
D = 768
MAX_POS = 2048
OFFSET = 2
N = B * S

NC, NS = 2, 16
NW = NC * NS
RPW = N // NW
CH = 32
NCH = RPW // CH
NBUF = 5

_mesh = plsc.VectorSubcoreMesh(core_axis_name="c", subcore_axis_name="s")


L = 16


@functools.partial(
    pl.kernel,
    mesh=_mesh,
    out_type=jax.ShapeDtypeStruct((N, D), jnp.float32),
    scratch_types=[
        pltpu.VMEM((B, RPW), jnp.int32),
    ]
    + [pltpu.VMEM((CH, D), jnp.float32) for _ in range(NBUF)]
    + [pltpu.SemaphoreType.DMA for _ in range(2 * NBUF)],
)
def _tok_kernel(ids_hbm, table_hbm, out_tok, idx_v, *bufs_and_sems):
    bufs = bufs_and_sems[:NBUF]
    gsems = bufs_and_sems[NBUF:2 * NBUF]
    ssems = bufs_and_sems[2 * NBUF:]
    wid = lax.axis_index("s") * NC + lax.axis_index("c")
    b = wid % B
    cb = wid // B
    base = b * S + cb * RPW

    pltpu.sync_copy(ids_hbm.at[:, pl.ds(cb * RPW, RPW)], idx_v)

    def gather(c):
        cps = []
        for k in range(CH // L):
            vec = idx_v[b, pl.ds(c * CH + k * L, L)]
            cp = pltpu.make_async_copy(
                table_hbm.at[vec],
                bufs[c % NBUF].at[pl.ds(k * L, L)],
                gsems[c % NBUF],
            )
            cp.start()
            cps.append(cp)
        return cps

    def store(c):
        return pltpu.make_async_copy(
            bufs[c % NBUF], out_tok.at[pl.ds(base + c * CH, CH)],
            ssems[c % NBUF])

    LA = 4
    pend = {c: gather(c) for c in range(min(LA, NCH))}
    stores = {}
    for c in range(NCH):
        for cp in pend.pop(c):
            cp.wait()
        stores[c] = store(c)
        stores[c].start()
        nxt = c + LA
        if nxt < NCH:
            old = nxt - NBUF
            if old >= 0:
                stores.pop(old).wait()
            pend[nxt] = gather(nxt)
    for c in sorted(stores):
        stores[c].wait()


def _pos_body(src_ref, out_ref, mask_ref):
    out_ref[0] = src_ref[pl.ds(OFFSET, S), :]

    @pl.when(pl.program_id(0) == 0)
    def _():
        mask_ref[...] = jnp.ones_like(mask_ref)


_pos_kernel = pl.pallas_call(
    _pos_body,
    grid=(B,),
    in_specs=[pl.BlockSpec((MAX_POS + OFFSET, D), lambda b: (0, 0))],
    out_specs=[
        pl.BlockSpec((1, S, D), lambda b: (b, 0, 0)),
        pl.BlockSpec((B, S), lambda b: (0, 0)),
    ],
    out_shape=[
        jax.ShapeDtypeStruct((B, S, D), jnp.float32),
        jax.ShapeDtypeStruct((B, S), jnp.float32),
    ],
)


def kernel(input_ids, embed_tokens, embed_positions):
    ids = input_ids.astype(jnp.int32)
    tok_flat = _tok_kernel(ids, embed_tokens)
    pos_embeds, attention_mask = _pos_kernel(embed_positions)
    inputs_embeds = tok_flat.reshape(B, S, D)
    return (inputs_embeds, attention_mask, pos_embeds)


# speedup vs baseline: 1.0210x; 1.0041x over previous
import functools

import jax
import jax.numpy as jnp
from jax import lax
from jax.experimental import pallas as pl
from jax.experimental.pallas import tpu as pltpu
from jax.experimental.pallas import tpu_sc as plsc

B = 4
S = 2048
D = 768
MAX_POS = 2048
OFFSET = 2
N = B * S

NC, NS = 2, 16
NW = NC * NS
RPW = N // NW
CH = 32
NCH = RPW // CH
NBUF = 5

_mesh = plsc.VectorSubcoreMesh(core_axis_name="c", subcore_axis_name="s")


L = 16


@functools.partial(
    pl.kernel,
    mesh=_mesh,
    out_type=jax.ShapeDtypeStruct((N, D), jnp.float32),
    scratch_types=[
        pltpu.VMEM((B, RPW), jnp.int32),
    ]
    + [pltpu.VMEM((CH, D), jnp.float32) for _ in range(NBUF)]
    + [pltpu.SemaphoreType.DMA for _ in range(2 * NBUF)],
)
def _tok_kernel(ids_hbm, table_hbm, out_tok, idx_v, *bufs_and_sems):
    bufs = bufs_and_sems[:NBUF]
    gsems = bufs_and_sems[NBUF:2 * NBUF]
    ssems = bufs_and_sems[2 * NBUF:]
    wid = lax.axis_index("s") * NC + lax.axis_index("c")
    b = wid % B
    cb = wid // B
    base = b * S + cb * RPW

    pltpu.sync_copy(ids_hbm.at[:, pl.ds(cb * RPW, RPW)], idx_v)

    def gather(c):
        cps = []
        for k in range(CH // L):
            vec = idx_v[b, pl.ds(c * CH + k * L, L)]
            cp = pltpu.make_async_copy(
                table_hbm.at[vec],
                bufs[c % NBUF].at[pl.ds(k * L, L)],
                gsems[c % NBUF],
            )
            cp.start()
            cps.append(cp)
        return cps

    def store(c):
        return pltpu.make_async_copy(
            bufs[c % NBUF], out_tok.at[pl.ds(base + c * CH, CH)],
            ssems[c % NBUF])

    LA = 3
    pend = {c: gather(c) for c in range(min(LA, NCH))}
    stores = {}
    for c in range(NCH):
        for cp in pend.pop(c):
            cp.wait()
        stores[c] = store(c)
        stores[c].start()
        nxt = c + LA
        if nxt < NCH:
            old = nxt - NBUF
            if old >= 0:
                stores.pop(old).wait()
            pend[nxt] = gather(nxt)
    for c in sorted(stores):
        stores[c].wait()


def _pos_body(src_ref, out_ref, mask_ref):
    out_ref[0] = src_ref[pl.ds(OFFSET, S), :]

    @pl.when(pl.program_id(0) == 0)
    def _():
        mask_ref[...] = jnp.ones_like(mask_ref)


_pos_kernel = pl.pallas_call(
    _pos_body,
    grid=(B,),
    in_specs=[pl.BlockSpec((MAX_POS + OFFSET, D), lambda b: (0, 0))],
    out_specs=[
        pl.BlockSpec((1, S, D), lambda b: (b, 0, 0)),
        pl.BlockSpec((B, S), lambda b: (0, 0)),
    ],
    out_shape=[
        jax.ShapeDtypeStruct((B, S, D), jnp.float32),
        jax.ShapeDtypeStruct((B, S), jnp.float32),
    ],
)


def kernel(input_ids, embed_tokens, embed_positions):
    ids = input_ids.astype(jnp.int32)
    tok_flat = _tok_kernel(ids, embed_tokens)
    pos_embeds, attention_mask = _pos_kernel(embed_positions)
    inputs_embeds = tok_flat.reshape(B, S, D)
    return (inputs_embeds, attention_mask, pos_embeds)
